# Initial kernel scaffold; baseline (speedup 1.0000x reference)
import jax
import jax.numpy as jnp
from jax.experimental import pallas as pl


def kernel(x, edge_index, edge_weight, W1, b1, W2, b2, fc1_W, fc1_b, fc2_W, fc2_b):
    raise NotImplementedError("write your pallas kernel here")


# speedup vs baseline: 5.5675x; 5.5675x over previous
import dataclasses
import functools

import jax
import jax.numpy as jnp
from jax import lax
from jax.experimental import pallas as pl
from jax.experimental.pallas import tpu as pltpu
from jax.experimental.pallas import tpu_sc as plsc

N = 10000
F = 128
NCORES = 2
NSUB = 16
NW = NCORES * NSUB
EPW = 10000
CHUNK = 80
NCHUNKS = EPW // CHUNK
RPT = 624
TAIL = N - NSUB * RPT
ZROWS = 48


def _spmm_sc(h, row_r, col_r, w_r):
    mesh = plsc.VectorSubcoreMesh(core_axis_name="c", subcore_axis_name="s")

    cp = pltpu.CompilerParams()
    if "needs_layout_passes" in pltpu.CompilerParams.__dataclass_fields__:
        cp = dataclasses.replace(cp, needs_layout_passes=False)

    @functools.partial(
        pl.kernel,
        compiler_params=cp,
        out_type=jax.ShapeDtypeStruct((NCORES, N, F), jnp.float32),
        mesh=mesh,
        scratch_types=[
            pltpu.VMEM_SHARED((N, F), jnp.float32),
            pltpu.VMEM((NCHUNKS, CHUNK), jnp.int32),
            pltpu.VMEM((EPW,), jnp.int32),
            pltpu.VMEM((EPW,), jnp.float32),
            pltpu.VMEM((CHUNK, F), jnp.float32),
            pltpu.SemaphoreType.DMA,
        ],
    )
    def k(h_hbm, row_hbm, col_hbm, w_hbm, out_hbm,
          acc, row_v, col_v, w_v, rows_v, sem):
        cid = lax.axis_index("c")
        sid = lax.axis_index("s")
        wid = cid * NSUB + sid

        zeros16 = jnp.zeros((16,), jnp.float32)

        @pl.loop(0, CHUNK)
        def _(i):
            for q in range(F // 16):
                rows_v[i, pl.ds(q * 16, 16)] = zeros16

        for t in range(RPT // ZROWS):
            pltpu.sync_copy(rows_v.at[pl.ds(0, ZROWS)],
                            acc.at[pl.ds(sid * RPT + t * ZROWS, ZROWS)])

        @pl.when(sid == NSUB - 1)
        def _():
            pltpu.sync_copy(rows_v.at[pl.ds(0, TAIL)],
                            acc.at[pl.ds(NSUB * RPT, TAIL)])

        pltpu.sync_copy(row_hbm.at[wid], row_v)
        pltpu.sync_copy(col_hbm.at[wid], col_v)
        pltpu.sync_copy(w_hbm.at[wid], w_v)

        plsc.subcore_barrier()

        @pl.loop(0, NCHUNKS)
        def _(j):
            pltpu.async_copy(h_hbm.at[col_v.at[pl.ds(j * CHUNK, CHUNK)]],
                             rows_v, sem).wait()
            for e in range(CHUNK):
                widx = jnp.full((16,), j * CHUNK + e, jnp.int32)
                wvec = plsc.load_gather(w_v, [widx])
                for q in range(F // 16):
                    sl = pl.ds(q * 16, 16)
                    rows_v[e, sl] = rows_v[e, sl] * wvec
            pltpu.sync_copy(rows_v, acc.at[row_v.at[j]], add=True)

        plsc.subcore_barrier()

        base = sid * RPT
        pltpu.sync_copy(acc.at[pl.ds(base, RPT)],
                        out_hbm.at[cid, pl.ds(base, RPT)])

        @pl.when(sid == NSUB - 1)
        def _():
            pltpu.sync_copy(acc.at[pl.ds(NSUB * RPT, TAIL)],
                            out_hbm.at[cid, pl.ds(NSUB * RPT, TAIL)])

    return k(h, row_r, col_r, w_r)


_BLK = 1000


def _mm_in(x, W):
    def body(x_ref, w_ref, o_ref):
        o_ref[...] = jnp.dot(x_ref[...], w_ref[...],
                             preferred_element_type=jnp.float32)

    return pl.pallas_call(
        body,
        grid=(N // _BLK,),
        in_specs=[pl.BlockSpec((_BLK, F), lambda i: (i, 0)),
                  pl.BlockSpec((F, F), lambda i: (0, 0))],
        out_specs=pl.BlockSpec((_BLK, F), lambda i: (i, 0)),
        out_shape=jax.ShapeDtypeStruct((N, F), jnp.float32),
    )(x, W)


def _mm_mid(p, b, W):
    def body(p_ref, b_ref, w_ref, o_ref):
        h = p_ref[0] + p_ref[1] + b_ref[...]
        o_ref[...] = jnp.dot(h, w_ref[...],
                             preferred_element_type=jnp.float32)

    return pl.pallas_call(
        body,
        grid=(N // _BLK,),
        in_specs=[pl.BlockSpec((NCORES, _BLK, F), lambda i: (0, i, 0)),
                  pl.BlockSpec((1, F), lambda i: (0, 0)),
                  pl.BlockSpec((F, F), lambda i: (0, 0))],
        out_specs=pl.BlockSpec((_BLK, F), lambda i: (i, 0)),
        out_shape=jax.ShapeDtypeStruct((N, F), jnp.float32),
    )(p, b, W)


def _head(p, b, fc1_W, fc1_b, fc2_W, fc2_b):
    H1 = fc1_W.shape[1]
    H2 = fc2_W.shape[1]

    def body(p_ref, b_ref, w1_ref, b1_ref, w2_ref, b2_ref, o_ref):
        z = p_ref[0] + p_ref[1] + b_ref[...]
        t = jnp.dot(z, w1_ref[...], preferred_element_type=jnp.float32)
        t = t + b1_ref[...]
        h3 = jnp.where(t > 0, t, jnp.exp(jnp.minimum(t, 0.0)) - 1.0)
        o_ref[...] = jnp.dot(h3, w2_ref[...],
                             preferred_element_type=jnp.float32) + b2_ref[...]

    return pl.pallas_call(
        body,
        grid=(N // _BLK,),
        in_specs=[pl.BlockSpec((NCORES, _BLK, F), lambda i: (0, i, 0)),
                  pl.BlockSpec((1, F), lambda i: (0, 0)),
                  pl.BlockSpec((F, H1), lambda i: (0, 0)),
                  pl.BlockSpec((1, H1), lambda i: (0, 0)),
                  pl.BlockSpec((H1, H2), lambda i: (0, 0)),
                  pl.BlockSpec((1, H2), lambda i: (0, 0))],
        out_specs=pl.BlockSpec((_BLK, H2), lambda i: (i, 0)),
        out_shape=jax.ShapeDtypeStruct((N, H2), jnp.float32),
    )(p, b, fc1_W, fc1_b, fc2_W, fc2_b)


def kernel(x, edge_index, edge_weight, W1, b1, W2, b2, fc1_W, fc1_b, fc2_W, fc2_b):
    ei = edge_index.astype(jnp.int32)
    row_r = ei[0].reshape(NW, NCHUNKS, CHUNK)
    col_r = ei[1].reshape(NW, EPW)
    w_r = edge_weight.reshape(NW, EPW)

    s1 = _mm_in(x, W1)
    p1 = _spmm_sc(s1, row_r, col_r, w_r)
    s2 = _mm_mid(p1, b1.reshape(1, F), W2)
    p2 = _spmm_sc(s2, row_r, col_r, w_r)
    return _head(p2, b2.reshape(1, F), fc1_W, fc1_b.reshape(1, -1),
                 fc2_W, fc2_b.reshape(1, -1))
